# baseline (device time: 24284 ns/iter reference)
import jax
import jax.numpy as jnp
from jax import lax
from jax.experimental import pallas as pl
from jax.experimental.pallas import tpu as pltpu

N_DEV = 32
BLK = 64
HALF = 32
GROUPS = 4
PER_G = N_DEV // GROUPS


def kernel(x, w_mat):
    m_glob, k_per = x.shape
    k_glob, n = w_mat.shape
    m_per = m_glob // N_DEV

    x2 = x.reshape(m_glob // 2, 2 * k_per)

    def body(x2_ref, w_hbm, out_ref, w_ref, asm_ref, ye_ref, yo_ref,
             send_sems, recv_sems, ready_sems, w_sems):
        me = lax.axis_index("i")

        bar = pltpu.get_barrier_semaphore()
        pl.semaphore_signal(bar, inc=1, device_id=(me,),
                            device_id_type=pl.DeviceIdType.MESH)
        pl.semaphore_wait(bar, 1)

        w_rows = k_glob // GROUPS
        for c in range(GROUPS):
            pltpu.make_async_copy(
                w_hbm.at[pl.ds(c * w_rows, w_rows), :],
                w_ref.at[pl.ds(c * w_rows, w_rows), :],
                w_sems.at[c],
            ).start()

        asm_ref[me] = x2_ref[pl.ds(me * HALF, HALF), :]

        for d in range(1, N_DEV):
            t = lax.rem(me + d, N_DEV)
            pltpu.make_async_remote_copy(
                src_ref=x2_ref.at[pl.ds(t * HALF, HALF), :],
                dst_ref=asm_ref.at[me],
                send_sem=send_sems.at[d],
                recv_sem=recv_sems.at[me],
                device_id=(t,),
                device_id_type=pl.DeviceIdType.MESH,
            ).start()

        for g in range(GROUPS):
            for s in range(g * PER_G, (g + 1) * PER_G):
                @pl.when(s != me)
                def _():
                    pltpu.make_async_remote_copy(
                        src_ref=x2_ref.at[pl.ds(s * HALF, HALF), :],
                        dst_ref=asm_ref.at[s],
                        send_sem=send_sems.at[1],
                        recv_sem=recv_sems.at[s],
                        device_id=(s,),
                        device_id_type=pl.DeviceIdType.MESH,
                    ).wait_recv()
            a3 = asm_ref[pl.ds(g * PER_G, PER_G)]
            a_e = jnp.transpose(a3[:, :, 0:BLK], (1, 0, 2)).reshape(
                HALF, PER_G * BLK
            )
            a_o = jnp.transpose(a3[:, :, BLK:2 * BLK], (1, 0, 2)).reshape(
                HALF, PER_G * BLK
            )
            pltpu.make_async_copy(
                w_hbm.at[pl.ds(g * w_rows, w_rows), :],
                w_ref.at[pl.ds(g * w_rows, w_rows), :],
                w_sems.at[g],
            ).wait()
            w_g = w_ref[pl.ds(g * PER_G * BLK, PER_G * BLK), :]
            pe = jnp.dot(a_e, w_g, preferred_element_type=jnp.float32)
            po = jnp.dot(a_o, w_g, preferred_element_type=jnp.float32)
            if g == 0:
                ye_ref[:, :] = pe
                yo_ref[:, :] = po
            else:
                ye_ref[:, :] += pe
                yo_ref[:, :] += po

        yy = jnp.concatenate([ye_ref[:, :], yo_ref[:, :]], axis=0)
        out_ref[:, :] = jnp.transpose(
            yy.reshape(2, HALF, n), (1, 0, 2)
        ).reshape(m_per, n)

        for d in range(1, N_DEV):
            t = lax.rem(me + d, N_DEV)
            pltpu.make_async_remote_copy(
                src_ref=x2_ref.at[pl.ds(t * HALF, HALF), :],
                dst_ref=asm_ref.at[me],
                send_sem=send_sems.at[d],
                recv_sem=recv_sems.at[me],
                device_id=(t,),
                device_id_type=pl.DeviceIdType.MESH,
            ).wait_send()

    return pl.pallas_call(
        body,
        out_shape=jax.ShapeDtypeStruct((m_per, n), jnp.float32),
        in_specs=[
            pl.BlockSpec(memory_space=pltpu.VMEM),
            pl.BlockSpec(memory_space=pltpu.MemorySpace.HBM),
        ],
        out_specs=pl.BlockSpec(memory_space=pltpu.VMEM),
        scratch_shapes=[
            pltpu.VMEM((k_glob, n), jnp.float32),
            pltpu.VMEM((N_DEV, HALF, 2 * BLK), jnp.float32),
            pltpu.VMEM((HALF, n), jnp.float32),
            pltpu.VMEM((HALF, n), jnp.float32),
            pltpu.SemaphoreType.DMA((N_DEV,)),
            pltpu.SemaphoreType.DMA((N_DEV,)),
            pltpu.SemaphoreType.REGULAR((N_DEV,)),
            pltpu.SemaphoreType.DMA((GROUPS,)),
        ],
        compiler_params=pltpu.CompilerParams(collective_id=0),
    )(x2, w_mat)


# device time: 22586 ns/iter; 1.0752x vs baseline; 1.0752x over previous
import jax
import jax.numpy as jnp
from jax import lax
from jax.experimental import pallas as pl
from jax.experimental.pallas import tpu as pltpu

N_DEV = 32
BLK = 64
HALF = 32
GROUPS = 4
PER_G = N_DEV // GROUPS


def kernel(x, w_mat):
    m_glob, k_per = x.shape
    k_glob, n = w_mat.shape
    m_per = m_glob // N_DEV

    x2 = x.reshape(m_glob // 2, 2 * k_per)

    def body(x2_ref, w_ref, out_ref, asm_ref, ye_ref, yo_ref,
             send_sems, recv_sems):
        me = lax.axis_index("i")

        bar = pltpu.get_barrier_semaphore()
        pl.semaphore_signal(bar, inc=1, device_id=(me,),
                            device_id_type=pl.DeviceIdType.MESH)
        pl.semaphore_wait(bar, 1)

        asm_ref[me] = x2_ref[pl.ds(me * HALF, HALF), :]

        for d in range(1, N_DEV):
            t = lax.rem(me + d, N_DEV)
            pltpu.make_async_remote_copy(
                src_ref=x2_ref.at[pl.ds(t * HALF, HALF), :],
                dst_ref=asm_ref.at[me],
                send_sem=send_sems.at[d],
                recv_sem=recv_sems.at[me],
                device_id=(t,),
                device_id_type=pl.DeviceIdType.MESH,
            ).start()

        for g in range(GROUPS):
            for s in range(g * PER_G, (g + 1) * PER_G):
                @pl.when(s != me)
                def _():
                    pltpu.make_async_remote_copy(
                        src_ref=x2_ref.at[pl.ds(s * HALF, HALF), :],
                        dst_ref=asm_ref.at[s],
                        send_sem=send_sems.at[1],
                        recv_sem=recv_sems.at[s],
                        device_id=(s,),
                        device_id_type=pl.DeviceIdType.MESH,
                    ).wait_recv()
            a3 = asm_ref[pl.ds(g * PER_G, PER_G)]
            a_e = jnp.transpose(a3[:, :, 0:BLK], (1, 0, 2)).reshape(
                HALF, PER_G * BLK
            )
            a_o = jnp.transpose(a3[:, :, BLK:2 * BLK], (1, 0, 2)).reshape(
                HALF, PER_G * BLK
            )
            w_g = w_ref[pl.ds(g * PER_G * BLK, PER_G * BLK), :]
            pe = jnp.dot(a_e, w_g, preferred_element_type=jnp.float32)
            po = jnp.dot(a_o, w_g, preferred_element_type=jnp.float32)
            if g == 0:
                ye_ref[:, :] = pe
                yo_ref[:, :] = po
            else:
                ye_ref[:, :] += pe
                yo_ref[:, :] += po

        yy = jnp.concatenate([ye_ref[:, :], yo_ref[:, :]], axis=0)
        out_ref[:, :] = jnp.transpose(
            yy.reshape(2, HALF, n), (1, 0, 2)
        ).reshape(m_per, n)

        for d in range(1, N_DEV):
            t = lax.rem(me + d, N_DEV)
            pltpu.make_async_remote_copy(
                src_ref=x2_ref.at[pl.ds(t * HALF, HALF), :],
                dst_ref=asm_ref.at[me],
                send_sem=send_sems.at[d],
                recv_sem=recv_sems.at[me],
                device_id=(t,),
                device_id_type=pl.DeviceIdType.MESH,
            ).wait_send()

    return pl.pallas_call(
        body,
        out_shape=jax.ShapeDtypeStruct((m_per, n), jnp.float32),
        in_specs=[
            pl.BlockSpec(memory_space=pltpu.VMEM),
            pl.BlockSpec(memory_space=pltpu.VMEM),
        ],
        out_specs=pl.BlockSpec(memory_space=pltpu.VMEM),
        scratch_shapes=[
            pltpu.VMEM((N_DEV, HALF, 2 * BLK), jnp.float32),
            pltpu.VMEM((HALF, n), jnp.float32),
            pltpu.VMEM((HALF, n), jnp.float32),
            pltpu.SemaphoreType.DMA((N_DEV,)),
            pltpu.SemaphoreType.DMA((N_DEV,)),
        ],
        compiler_params=pltpu.CompilerParams(collective_id=0),
    )(x2, w_mat)
